# baseline (speedup 1.0000x reference)
import functools

import jax
import jax.numpy as jnp
from jax import lax
from jax.experimental import pallas as pl
from jax.experimental.pallas import tpu as pltpu
from jax.experimental.pallas import tpu_sc as plsc

N = 10000
E = 320000
HID = 64
NC, NS = 2, 16
NW = NC * NS
EPT = E // NW
CH = 64
NCHUNK = 157
EPTP = NCHUNK * CH
AW = 80
NP = 10240
RPT = NP // NS
BLK = 2000
GRID = N // BLK
NBUF = 4
NGRP = 39


def _enc_body(x_ref, we1_ref, be1_ref, we2_ref, be2_ref, wm1_ref, bm1_ref,
              h_ref, t_ref):
    i = pl.program_id(0)
    x = x_ref[...]
    h1 = jnp.maximum(
        jnp.dot(x, we1_ref[...], preferred_element_type=jnp.float32)
        + be1_ref[...], 0.0)
    h = (jnp.dot(h1, we2_ref[...], preferred_element_type=jnp.float32)
         + be2_ref[...])
    h_ref[...] = h
    wm1 = wm1_ref[...]
    is_a = i < GRID
    wsel = jnp.where(is_a, wm1[:HID], wm1[HID:])
    bsel = bm1_ref[...] * jnp.where(is_a, 1.0, 0.0)
    t_ref[...] = (jnp.dot(h, wsel, preferred_element_type=jnp.float32)
                  + bsel)


def _stage1(x, we1, be1, we2, be2, wm1, bm1):
    full = lambda r, c: pl.BlockSpec((r, c), lambda i: (0, 0))
    return pl.pallas_call(
        _enc_body,
        grid=(2 * GRID,),
        in_specs=[
            pl.BlockSpec((BLK, 128), lambda i: (i % GRID, 0)),
            full(128, HID), full(1, HID),
            full(HID, HID), full(1, HID),
            full(2 * HID, HID), full(1, HID),
        ],
        out_specs=[
            pl.BlockSpec((BLK, HID), lambda i: (i % GRID, 0)),
            pl.BlockSpec((BLK, HID), lambda i: (i, 0)),
        ],
        out_shape=[
            jax.ShapeDtypeStruct((N, HID), jnp.float32),
            jax.ShapeDtypeStruct((2 * N, HID), jnp.float32),
        ],
    )(x, we1, be1, we2, be2, wm1, bm1)


_mesh = plsc.VectorSubcoreMesh(core_axis_name="c", subcore_axis_name="s")


@functools.partial(
    pl.kernel,
    out_type=[jax.ShapeDtypeStruct((NP, AW), jnp.float32),
              jax.ShapeDtypeStruct((NP, AW), jnp.float32)],
    mesh=_mesh,
    compiler_params=pltpu.CompilerParams(use_tc_tiling_on_sc=False),
    scratch_types=[
        pltpu.VMEM((NCHUNK, CH), jnp.int32),
        pltpu.VMEM((NCHUNK, CH), jnp.int32),
        pltpu.VMEM((NBUF, 2 * CH), jnp.int32),
        [pltpu.VMEM((2 * CH, HID), jnp.float32)] * NBUF,
        [pltpu.VMEM((CH, AW), jnp.float32)] * NBUF,
        pltpu.VMEM_SHARED((NP, AW), jnp.float32),
        [pltpu.SemaphoreType.DMA] * NBUF,
        [pltpu.SemaphoreType.DMA] * NBUF,
    ],
)
def _sc_agg(tab_hbm, ridx_hbm, cidx_hbm, zeros_hbm, p0_hbm, p1_hbm,
            ridx_v, cidx_v, gidx_v, rab, out, acc_sh, sa, ss):
    c = lax.axis_index("c")
    s = lax.axis_index("s")
    wid = c * NS + s

    pltpu.sync_copy(zeros_hbm.at[pl.ds(s * RPT, RPT)],
                    acc_sh.at[pl.ds(s * RPT, RPT)])
    pltpu.sync_copy(ridx_hbm.at[pl.ds(wid * NCHUNK, NCHUNK)], ridx_v)
    pltpu.sync_copy(cidx_hbm.at[pl.ds(wid * NCHUNK, NCHUNK)], cidx_v)

    e0 = jnp.where(lax.iota(jnp.int32, 16) == 0, 1.0, 0.0)

    def _init_row(r, carry):
        for b in range(NBUF):
            out[b][r, pl.ds(HID, 16)] = e0
        return carry

    lax.fori_loop(0, CH, _init_row, 0)
    plsc.subcore_barrier()

    def _issue_gather(j, b):
        for cc in range(CH // 16):
            sl = pl.ds(cc * 16, 16)
            gidx_v[b, sl] = ridx_v[j, sl]
            gidx_v[b, pl.ds(CH + cc * 16, 16)] = jnp.minimum(
                cidx_v[j, sl] + N, 2 * N - 1)
        pltpu.async_copy(tab_hbm.at[gidx_v.at[b]], rab[b], sa[b])

    def _wait_gather(j, b):
        pltpu.make_async_copy(tab_hbm.at[gidx_v.at[b]], rab[b],
                              sa[b]).wait()

    def _compute(b):
        @plsc.parallel_loop(0, CH, 1, unroll=8)
        def _row(r):
            for cc in range(HID // 16):
                sl = pl.ds(cc * 16, 16)
                out[b][r, sl] = jnp.maximum(
                    rab[b][r, sl] + rab[b][CH + r, sl], 0.0)

    def _issue_scatter(j, b):
        pltpu.async_copy(out[b], acc_sh.at[cidx_v.at[j]], ss[b], add=True)

    def _drain_scatter(j, b):
        pltpu.make_async_copy(out[b], acc_sh.at[cidx_v.at[j]],
                              ss[b]).wait()

    for b in range(NBUF):
        _issue_gather(b, b)

    def _group(g, carry):
        for b in range(NBUF):
            j = NBUF * g + b
            _wait_gather(j, b)

            @pl.when(g > 0)
            def _():
                _drain_scatter(j, b)

            _compute(b)
            _issue_scatter(j, b)

            @pl.when(j + NBUF < NCHUNK)
            def _():
                _issue_gather(j + NBUF, b)
        return carry

    lax.fori_loop(0, NGRP, _group, 0)

    jt = NCHUNK - 1
    _wait_gather(jt, 0)
    _drain_scatter(jt, 0)
    _compute(0)
    _issue_scatter(jt, 0)
    for b in range(NBUF):
        _drain_scatter(jt, b)

    plsc.subcore_barrier()

    @pl.when(c == 0)
    def _():
        pltpu.sync_copy(acc_sh.at[pl.ds(s * RPT, RPT)],
                        p0_hbm.at[pl.ds(s * RPT, RPT)])

    @pl.when(c == 1)
    def _():
        pltpu.sync_copy(acc_sh.at[pl.ds(s * RPT, RPT)],
                        p1_hbm.at[pl.ds(s * RPT, RPT)])


def _upd_body(h_ref, p0_ref, p1_ref, wm2_ref, bm2_ref, wu1_ref, bu1_ref,
              wu2_ref, bu2_ref, wt1_ref, bt1_ref, wt2_ref, bt2_ref,
              wt3_ref, bt3_ref, h2_ref, tac_ref):
    p = p0_ref[...] + p1_ref[...]
    srelu = p[:, :HID]
    cnt = p[:, HID:HID + 1]
    pos = (cnt > 0.0).astype(jnp.float32)
    inv = pos / jnp.maximum(cnt, 1.0)
    agg = (jnp.dot(srelu * inv, wm2_ref[...],
                   preferred_element_type=jnp.float32)
           + pos * bm2_ref[...])
    h = h_ref[...]
    wu1 = wu1_ref[...]
    u = jnp.maximum(
        jnp.dot(h, wu1[:HID], preferred_element_type=jnp.float32)
        + jnp.dot(agg, wu1[HID:], preferred_element_type=jnp.float32)
        + bu1_ref[...], 0.0)
    h2 = (jnp.dot(u, wu2_ref[...], preferred_element_type=jnp.float32)
          + bu2_ref[...])
    h2_ref[...] = h2
    t = jnp.maximum(
        jnp.dot(h2, wt1_ref[...], preferred_element_type=jnp.float32)
        + bt1_ref[...], 0.0)
    t = jnp.maximum(
        jnp.dot(t, wt2_ref[...], preferred_element_type=jnp.float32)
        + bt2_ref[...], 0.0)
    tac_ref[...] = (jnp.dot(t, wt3_ref[...],
                            preferred_element_type=jnp.float32)
                    + bt3_ref[...])


def _stage2(h, p0, p1, wm2, bm2, wu1, bu1, wu2, bu2,
            wt1, bt1, wt2, bt2, wt3, bt3):
    full = lambda r, c: pl.BlockSpec((r, c), lambda i: (0, 0))
    return pl.pallas_call(
        _upd_body,
        grid=(GRID,),
        in_specs=[
            pl.BlockSpec((BLK, HID), lambda i: (i, 0)),
            pl.BlockSpec((BLK, AW), lambda i: (i, 0)),
            pl.BlockSpec((BLK, AW), lambda i: (i, 0)),
            full(HID, HID), full(1, HID),
            full(2 * HID, HID), full(1, HID),
            full(HID, 32), full(1, 32),
            full(32, HID), full(1, HID),
            full(HID, 16), full(1, 16),
            full(16, 4), full(1, 4),
        ],
        out_specs=[
            pl.BlockSpec((BLK, 32), lambda i: (i, 0)),
            pl.BlockSpec((BLK, 4), lambda i: (i, 0)),
        ],
        out_shape=[
            jax.ShapeDtypeStruct((N, 32), jnp.float32),
            jax.ShapeDtypeStruct((N, 4), jnp.float32),
        ],
    )(h, p0, p1, wm2, bm2, wu1, bu1, wu2, bu2,
      wt1, bt1, wt2, bt2, wt3, bt3)


def kernel(node_features, edge_indices,
           W_enc1, b_enc1, W_enc2, b_enc2,
           W_msg1, b_msg1, W_msg2, b_msg2,
           W_upd1, b_upd1, W_upd2, b_upd2,
           W_tac1, b_tac1, W_tac2, b_tac2, W_tac3, b_tac3):
    pad = EPTP - EPT
    rr = edge_indices[0].astype(jnp.int32).reshape(NW, EPT)
    cc = edge_indices[1].astype(jnp.int32).reshape(NW, EPT)
    ridx = jnp.concatenate(
        [rr, jnp.zeros((NW, pad), jnp.int32)], axis=1).reshape(
            NW * NCHUNK, CH)
    cidx = jnp.concatenate(
        [cc, jnp.full((NW, pad), NP - 1, jnp.int32)], axis=1).reshape(
            NW * NCHUNK, CH)

    r2 = lambda v: v.reshape(1, -1)
    h, tab = _stage1(node_features, W_enc1, r2(b_enc1), W_enc2, r2(b_enc2),
                     W_msg1, r2(b_msg1))

    zeros = jnp.zeros((NP, AW), jnp.float32)
    p0, p1 = _sc_agg(tab, ridx, cidx, zeros)

    h2, tactical = _stage2(h, p0, p1,
                           W_msg2, r2(b_msg2), W_upd1, r2(b_upd1),
                           W_upd2, r2(b_upd2), W_tac1, r2(b_tac1),
                           W_tac2, r2(b_tac2), W_tac3, r2(b_tac3))
    return (h2, tactical)


# speedup vs baseline: 1.4249x; 1.4249x over previous
import functools

import numpy as np

import jax
import jax.numpy as jnp
from jax import lax
from jax.experimental import pallas as pl
from jax.experimental.pallas import tpu as pltpu
from jax.experimental.pallas import tpu_sc as plsc

N = 10000
E = 320000
HID = 64
NC, NS = 2, 16
NW = NC * NS
EPT = E // NW
CH = 80
NCHUNK = 125
EPTP = NCHUNK * CH
AW = 80
NP = 10240
RPT = NP // NS
BLK = 2000
GRID = N // BLK


def _enc_body(x_ref, we1_ref, be1_ref, we2_ref, be2_ref, wm1_ref, bm1_ref,
              h_ref, a_ref, b_ref):
    x = x_ref[...]
    h1 = jnp.maximum(
        jnp.dot(x, we1_ref[...], preferred_element_type=jnp.float32)
        + be1_ref[...], 0.0)
    h = (jnp.dot(h1, we2_ref[...], preferred_element_type=jnp.float32)
         + be2_ref[...])
    h_ref[...] = h
    wm1 = wm1_ref[...]
    a_ref[...] = (jnp.dot(h, wm1[:HID], preferred_element_type=jnp.float32)
                  + bm1_ref[...]).astype(jnp.bfloat16)
    b_ref[...] = jnp.dot(h, wm1[HID:],
                         preferred_element_type=jnp.float32).astype(
                             jnp.bfloat16)


def _stage1(x, we1, be1, we2, be2, wm1, bm1):
    full = lambda r, c: pl.BlockSpec((r, c), lambda i: (0, 0))
    return pl.pallas_call(
        _enc_body,
        grid=(GRID,),
        in_specs=[
            pl.BlockSpec((BLK, 128), lambda i: (i, 0)),
            full(128, HID), full(1, HID),
            full(HID, HID), full(1, HID),
            full(2 * HID, HID), full(1, HID),
        ],
        out_specs=[
            pl.BlockSpec((BLK, HID), lambda i: (i, 0)),
            pl.BlockSpec((BLK, HID), lambda i: (i, 0)),
            pl.BlockSpec((BLK, HID), lambda i: (i, 0)),
        ],
        out_shape=[
            jax.ShapeDtypeStruct((N, HID), jnp.float32),
            jax.ShapeDtypeStruct((N, HID), jnp.bfloat16),
            jax.ShapeDtypeStruct((N, HID), jnp.bfloat16),
        ],
    )(x, we1, be1, we2, be2, wm1, bm1)


_mesh = plsc.VectorSubcoreMesh(core_axis_name="c", subcore_axis_name="s")


NBUF = 4
NGRP = 31


@functools.partial(
    pl.kernel,
    out_type=[jax.ShapeDtypeStruct((NP, AW), jnp.float32),
              jax.ShapeDtypeStruct((NP, AW), jnp.float32)],
    mesh=_mesh,
    compiler_params=pltpu.CompilerParams(use_tc_tiling_on_sc=False,
                                         needs_layout_passes=False),
    scratch_types=[
        pltpu.VMEM((NCHUNK, CH), jnp.int32),
        pltpu.VMEM((NCHUNK, CH), jnp.int32),
        [pltpu.VMEM((CH, HID), jnp.bfloat16)] * NBUF,
        [pltpu.VMEM((CH, HID), jnp.bfloat16)] * NBUF,
        [pltpu.VMEM((CH, AW), jnp.float32)] * NBUF,
        pltpu.VMEM_SHARED((NP, AW), jnp.float32),
        [pltpu.SemaphoreType.DMA] * NBUF,
        [pltpu.SemaphoreType.DMA] * NBUF,
        [pltpu.SemaphoreType.DMA] * NBUF,
    ],
)
def _sc_agg(a_hbm, b_hbm, idx_hbm, zeros_hbm, p0_hbm, p1_hbm,
            ridx_v, cidx_v, ra, rb, out, acc_sh, sa, sb, ss):
    c = lax.axis_index("c")
    s = lax.axis_index("s")
    wid = c * NS + s

    pltpu.sync_copy(zeros_hbm.at[pl.ds(s * RPT, RPT)],
                    acc_sh.at[pl.ds(s * RPT, RPT)])
    pltpu.sync_copy(idx_hbm.at[0, pl.ds(wid * NCHUNK, NCHUNK)], ridx_v)
    pltpu.sync_copy(idx_hbm.at[1, pl.ds(wid * NCHUNK, NCHUNK)], cidx_v)

    e0 = jnp.where(lax.iota(jnp.int32, 16) == 0, 1.0, 0.0)

    def _init_row(r, carry):
        for b in range(NBUF):
            out[b][r, pl.ds(HID, 16)] = e0
        return carry

    lax.fori_loop(0, CH, _init_row, 0)
    plsc.subcore_barrier()

    def _issue_gathers(j, b):
        pltpu.async_copy(a_hbm.at[ridx_v.at[j]], ra[b], sa[b])
        pltpu.async_copy(b_hbm.at[cidx_v.at[j]], rb[b], sb[b])

    def _wait_gathers(j, b):
        pltpu.make_async_copy(a_hbm.at[ridx_v.at[j]], ra[b], sa[b]).wait()
        pltpu.make_async_copy(b_hbm.at[cidx_v.at[j]], rb[b], sb[b]).wait()

    def _compute(b):
        @plsc.parallel_loop(0, CH, 1, unroll=8)
        def _row(r):
            for hh in range(2):
                xa, xb = plsc.unpack(
                    ra[b][r, pl.ds(hh * 32, 32)],
                    format=plsc.PackFormat.INTERLEAVED)
                ya, yb = plsc.unpack(
                    rb[b][r, pl.ds(hh * 32, 32)],
                    format=plsc.PackFormat.INTERLEAVED)
                out[b][r, pl.ds(hh * 32, 16)] = jnp.maximum(xa + ya, 0.0)
                out[b][r, pl.ds(hh * 32 + 16, 16)] = jnp.maximum(
                    xb + yb, 0.0)

    def _issue_scatter(j, b):
        pltpu.async_copy(out[b], acc_sh.at[cidx_v.at[j]], ss[b], add=True)

    def _drain_scatter(j, b):
        pltpu.make_async_copy(out[b], acc_sh.at[cidx_v.at[j]],
                              ss[b]).wait()

    for b in range(NBUF):
        _issue_gathers(b, b)

    def _group(g, carry):
        for b in range(NBUF):
            j = NBUF * g + b
            _wait_gathers(j, b)

            @pl.when(g > 0)
            def _():
                _drain_scatter(j, b)

            _compute(b)
            _issue_scatter(j, b)

            @pl.when(j + NBUF < NCHUNK)
            def _():
                _issue_gathers(j + NBUF, b)
        return carry

    lax.fori_loop(0, NGRP, _group, 0)

    jt = NCHUNK - 1
    _wait_gathers(jt, 0)
    _drain_scatter(jt, 0)
    _compute(0)
    _issue_scatter(jt, 0)
    for b in range(NBUF):
        _drain_scatter(jt, b)

    plsc.subcore_barrier()

    @pl.when(c == 0)
    def _():
        pltpu.sync_copy(acc_sh.at[pl.ds(s * RPT, RPT)],
                        p0_hbm.at[pl.ds(s * RPT, RPT)])

    @pl.when(c == 1)
    def _():
        pltpu.sync_copy(acc_sh.at[pl.ds(s * RPT, RPT)],
                        p1_hbm.at[pl.ds(s * RPT, RPT)])


def _upd_body(h_ref, p0_ref, p1_ref, wm2_ref, bm2_ref, wu1_ref, bu1_ref,
              wu2_ref, bu2_ref, wt1_ref, bt1_ref, wt2_ref, bt2_ref,
              wt3_ref, bt3_ref, h2_ref, tac_ref):
    p = p0_ref[...] + p1_ref[...]
    srelu = p[:, :HID]
    cnt = p[:, HID:HID + 1]
    pos = (cnt > 0.0).astype(jnp.float32)
    inv = pos / jnp.maximum(cnt, 1.0)
    agg = (jnp.dot(srelu * inv, wm2_ref[...],
                   preferred_element_type=jnp.float32)
           + pos * bm2_ref[...])
    h = h_ref[...]
    wu1 = wu1_ref[...]
    u = jnp.maximum(
        jnp.dot(h, wu1[:HID], preferred_element_type=jnp.float32)
        + jnp.dot(agg, wu1[HID:], preferred_element_type=jnp.float32)
        + bu1_ref[...], 0.0)
    h2 = (jnp.dot(u, wu2_ref[...], preferred_element_type=jnp.float32)
          + bu2_ref[...])
    h2_ref[...] = h2
    t = jnp.maximum(
        jnp.dot(h2, wt1_ref[...], preferred_element_type=jnp.float32)
        + bt1_ref[...], 0.0)
    t = jnp.maximum(
        jnp.dot(t, wt2_ref[...], preferred_element_type=jnp.float32)
        + bt2_ref[...], 0.0)
    tac_ref[...] = (jnp.dot(t, wt3_ref[...],
                            preferred_element_type=jnp.float32)
                    + bt3_ref[...])


def _stage2(h, p0, p1, wm2, bm2, wu1, bu1, wu2, bu2,
            wt1, bt1, wt2, bt2, wt3, bt3):
    full = lambda r, c: pl.BlockSpec((r, c), lambda i: (0, 0))
    return pl.pallas_call(
        _upd_body,
        grid=(GRID,),
        in_specs=[
            pl.BlockSpec((BLK, HID), lambda i: (i, 0)),
            pl.BlockSpec((BLK, AW), lambda i: (i, 0)),
            pl.BlockSpec((BLK, AW), lambda i: (i, 0)),
            full(HID, HID), full(1, HID),
            full(2 * HID, HID), full(1, HID),
            full(HID, 32), full(1, 32),
            full(32, HID), full(1, HID),
            full(HID, 16), full(1, 16),
            full(16, 4), full(1, 4),
        ],
        out_specs=[
            pl.BlockSpec((BLK, 32), lambda i: (i, 0)),
            pl.BlockSpec((BLK, 4), lambda i: (i, 0)),
        ],
        out_shape=[
            jax.ShapeDtypeStruct((N, 32), jnp.float32),
            jax.ShapeDtypeStruct((N, 4), jnp.float32),
        ],
    )(h, p0, p1, wm2, bm2, wu1, bu1, wu2, bu2,
      wt1, bt1, wt2, bt2, wt3, bt3)


_PERM = np.concatenate([np.concatenate([blk + np.arange(0, 32, 2),
                                        blk + np.arange(1, 32, 2)])
                        for blk in (0, 32)])


def kernel(node_features, edge_indices,
           W_enc1, b_enc1, W_enc2, b_enc2,
           W_msg1, b_msg1, W_msg2, b_msg2,
           W_upd1, b_upd1, W_upd2, b_upd2,
           W_tac1, b_tac1, W_tac2, b_tac2, W_tac3, b_tac3):
    idx = edge_indices.astype(jnp.int32).reshape(2, NW * NCHUNK, CH)

    r2 = lambda v: v.reshape(1, -1)
    h, a, b = _stage1(node_features, W_enc1, r2(b_enc1), W_enc2, r2(b_enc2),
                      W_msg1, r2(b_msg1))
    wm2p = W_msg2[_PERM]

    zeros = jnp.zeros((NP, AW), jnp.float32)
    p0, p1 = _sc_agg(a, b, idx, zeros)

    h2, tactical = _stage2(h, p0, p1,
                           wm2p, r2(b_msg2), W_upd1, r2(b_upd1),
                           W_upd2, r2(b_upd2), W_tac1, r2(b_tac1),
                           W_tac2, r2(b_tac2), W_tac3, r2(b_tac3))
    return (h2, tactical)
